# Initial kernel scaffold; baseline (speedup 1.0000x reference)
import jax
import jax.numpy as jnp
from jax import lax
from jax.experimental import pallas as pl
from jax.experimental.pallas import tpu as pltpu
from jax.experimental.pallas import tpu_sc as plsc

MAXLEN_USED = 200
EMBED = 128
BATCH = 4096
LANES = 16

NUM_CORES = 2
NUM_SUBCORES = 16
NW = NUM_CORES * NUM_SUBCORES
SEQ_PER_W = BATCH // NW

CHUNK_A = 104
CHUNK_B = 96


def _body(x_hbm, tok_hbm, pos_hbm, out_hbm, idx_v, pos_v, buf_v, gsem):
    wid = lax.axis_index("s") * NUM_CORES + lax.axis_index("c")
    seq0 = wid * SEQ_PER_W

    pltpu.sync_copy(pos_hbm.at[pl.ds(0, MAXLEN_USED)], pos_v)
    pltpu.sync_copy(
        x_hbm.at[pl.ds(seq0 * MAXLEN_USED, SEQ_PER_W * MAXLEN_USED)], idx_v
    )

    @pl.loop(0, SEQ_PER_W)
    def _seq(k):
        off = k * MAXLEN_USED
        cp_a = pltpu.async_copy(
            tok_hbm.at[idx_v.at[pl.ds(off, CHUNK_A)]],
            buf_v.at[pl.ds(0, CHUNK_A)],
            gsem,
        )
        cp_b = pltpu.async_copy(
            tok_hbm.at[idx_v.at[pl.ds(off + CHUNK_A, CHUNK_B)]],
            buf_v.at[pl.ds(CHUNK_A, CHUNK_B)],
            gsem,
        )
        cp_a.wait()
        cp_b.wait()

        @pl.loop(0, MAXLEN_USED)
        def _row(r):
            for c in range(EMBED // LANES):
                sl = pl.ds(c * LANES, LANES)
                buf_v[r, sl] = buf_v[r, sl] + pos_v[r, sl]

        pltpu.sync_copy(buf_v, out_hbm.at[seq0 + k])


def kernel(x, token_table, pos_table):
    x_flat = x.reshape(-1).astype(jnp.int32)
    mesh = plsc.VectorSubcoreMesh(
        core_axis_name="c", subcore_axis_name="s"
    )
    f = pl.kernel(
        _body,
        out_type=jax.ShapeDtypeStruct((BATCH, MAXLEN_USED, EMBED), jnp.float32),
        mesh=mesh,
        scratch_types=[
            pltpu.VMEM((SEQ_PER_W * MAXLEN_USED,), jnp.int32),
            pltpu.VMEM((MAXLEN_USED, EMBED), jnp.float32),
            pltpu.VMEM((MAXLEN_USED, EMBED), jnp.float32),
            pltpu.SemaphoreType.DMA,
        ],
    )
    return f(x_flat, token_table, pos_table)


# speedup vs baseline: 9.0009x; 9.0009x over previous
import jax
import jax.numpy as jnp
from jax import lax
from jax.experimental import pallas as pl
from jax.experimental.pallas import tpu as pltpu
from jax.experimental.pallas import tpu_sc as plsc

MAXLEN_USED = 200
EMBED = 128
BATCH = 4096
LANES = 16

NUM_CORES = 2
NUM_SUBCORES = 16
NW = NUM_CORES * NUM_SUBCORES
SEQ_PER_W = BATCH // NW
NBUF = 3

CHUNK_A = 104
CHUNK_B = 96


def _body(x_hbm, tok_hbm, pos_hbm, out_hbm,
          idx_v, pos_v, b0, b1, b2, g0, g1, g2, s0, s1, s2):
    bufs = (b0, b1, b2)
    gs = (g0, g1, g2)
    ss = (s0, s1, s2)
    wid = lax.axis_index("s") * NUM_CORES + lax.axis_index("c")
    seq0 = wid * SEQ_PER_W

    pltpu.sync_copy(pos_hbm.at[pl.ds(0, MAXLEN_USED)], pos_v)
    pltpu.sync_copy(
        x_hbm.at[pl.ds(seq0 * MAXLEN_USED, SEQ_PER_W * MAXLEN_USED)], idx_v
    )

    def start_gather(k, b):
        off = k * MAXLEN_USED
        pltpu.async_copy(
            tok_hbm.at[idx_v.at[pl.ds(off, CHUNK_A)]],
            bufs[b].at[pl.ds(0, CHUNK_A)], gs[b])
        pltpu.async_copy(
            tok_hbm.at[idx_v.at[pl.ds(off + CHUNK_A, CHUNK_B)]],
            bufs[b].at[pl.ds(CHUNK_A, CHUNK_B)], gs[b])

    def wait_gather(b):
        pltpu.make_async_copy(
            tok_hbm.at[pl.ds(0, MAXLEN_USED)], bufs[b], gs[b]).wait()

    def start_store(k, b):
        pltpu.async_copy(bufs[b], out_hbm.at[seq0 + k], ss[b])

    def wait_store(b):
        pltpu.make_async_copy(bufs[b], out_hbm.at[0], ss[b]).wait()

    def add_pos(b):
        buf = bufs[b]

        @plsc.parallel_loop(0, MAXLEN_USED, 1, unroll=2)
        def _row(r):
            for c in range(EMBED // LANES):
                sl = pl.ds(c * LANES, LANES)
                buf[r, sl] = buf[r, sl] + pos_v[r, sl]

    start_gather(0, 0)
    start_gather(1, 1)
    start_gather(2, 2)

    wait_gather(0)
    add_pos(0)
    start_store(0, 0)

    wait_gather(1)
    add_pos(1)
    wait_store(0)
    start_gather(3, 0)
    start_store(1, 1)

    @pl.loop(2, SEQ_PER_W, step=NBUF)
    def _main(k0):
        for db in range(NBUF):
            k = k0 + db
            b = (2 + db) % NBUF
            nb = (b + 2) % NBUF
            wait_gather(b)
            add_pos(b)

            @pl.when(k + 2 < SEQ_PER_W)
            def _issue():
                wait_store(nb)
                start_gather(k + 2, nb)

            start_store(k, b)

    for b in range(NBUF):
        wait_store(b)


def kernel(x, token_table, pos_table):
    x_flat = x.reshape(-1).astype(jnp.int32)
    mesh = plsc.VectorSubcoreMesh(
        core_axis_name="c", subcore_axis_name="s",
        num_cores=NUM_CORES, num_subcores=NUM_SUBCORES,
    )
    f = pl.kernel(
        _body,
        out_type=jax.ShapeDtypeStruct((BATCH, MAXLEN_USED, EMBED), jnp.float32),
        mesh=mesh,
        scratch_types=[
            pltpu.VMEM((SEQ_PER_W * MAXLEN_USED,), jnp.int32),
            pltpu.VMEM((MAXLEN_USED, EMBED), jnp.float32),
            pltpu.VMEM((MAXLEN_USED, EMBED), jnp.float32),
            pltpu.VMEM((MAXLEN_USED, EMBED), jnp.float32),
            pltpu.VMEM((MAXLEN_USED, EMBED), jnp.float32),
            pltpu.SemaphoreType.DMA,
            pltpu.SemaphoreType.DMA,
            pltpu.SemaphoreType.DMA,
            pltpu.SemaphoreType.DMA,
            pltpu.SemaphoreType.DMA,
            pltpu.SemaphoreType.DMA,
        ],
    )
    return f(x_flat, token_table, pos_table)
